# baseline (speedup 1.0000x reference)
import functools

import jax
import jax.numpy as jnp
from jax import lax
from jax.experimental import pallas as pl
from jax.experimental.pallas import tpu as pltpu
from jax.experimental.pallas import tpu_sc as plsc

BATCH = 1024
WORDS = 200
COORDS = 64
NUM_WORKERS = 32
ROWS_TOTAL = BATCH * WORDS
ROWS_PER_W = ROWS_TOTAL // NUM_WORKERS
IDX_MINOR = 128
CHUNK = 256
IDX_ROWS = CHUNK // IDX_MINOR
N_CHUNKS = ROWS_PER_W // CHUNK


def kernel(sequence_bw, vocab_table_tc, pos_table_wc):
    seq_flat = sequence_bw.reshape(ROWS_TOTAL)
    mesh = plsc.VectorSubcoreMesh(core_axis_name="c", subcore_axis_name="s")

    @functools.partial(
        pl.kernel,
        out_type=jax.ShapeDtypeStruct((ROWS_TOTAL, COORDS), jnp.float32),
        mesh=mesh,
        scratch_types=[
            pltpu.VMEM((CHUNK,), jnp.int32),
            pltpu.VMEM((CHUNK, COORDS), jnp.float32),
            pltpu.VMEM((WORDS, COORDS), jnp.float32),
            pltpu.SemaphoreType.DMA,
        ],
        compiler_params=pltpu.CompilerParams(use_tc_tiling_on_sc=False),
    )
    def sc_kernel(seq_hbm, table_hbm, pos_hbm, out_hbm, idx_v, rows_v, pos_v, sem):
        wid = lax.axis_index("s") * 2 + lax.axis_index("c")
        pltpu.sync_copy(pos_hbm, pos_v)
        base0 = wid * ROWS_PER_W

        @pl.loop(0, N_CHUNKS)
        def _chunk(j):
            base = base0 + j * CHUNK
            pltpu.sync_copy(seq_hbm.at[pl.ds(base, CHUNK)], idx_v)
            for g in range(IDX_ROWS):
                pltpu.async_copy(
                    table_hbm.at[idx_v.at[pl.ds(g * IDX_MINOR, IDX_MINOR)]],
                    rows_v.at[pl.ds(g * IDX_MINOR, IDX_MINOR)],
                    sem,
                ).wait()

            @pl.loop(0, CHUNK)
            def _row(i):
                w = lax.rem(base + i, WORDS)
                for c in range(COORDS // 16):
                    plsc.addupdate(
                        rows_v.at[i, pl.ds(c * 16, 16)],
                        pos_v[w, pl.ds(c * 16, 16)],
                    )

            pltpu.sync_copy(rows_v, out_hbm.at[pl.ds(base, CHUNK)])

    out = sc_kernel(seq_flat, vocab_table_tc, pos_table_wc)
    return out.reshape(BATCH, WORDS, COORDS)
